# baseline (speedup 1.0000x reference)
import jax
import jax.numpy as jnp
from jax import lax
from jax.experimental import pallas as pl
from jax.experimental.pallas import tpu as pltpu

B = 8; S = 512; H = 128; INTENT = 128; SLOT = 512
GHD = 16; GOD = 128; NHEAD = 4; TOPK = 3; WINDOW = 2; ALPHA = 0.2
N = S + INTENT + SLOT
SI = S + INTENT


def _topk_onehot(x, k):
    rows, cols = x.shape
    ci = lax.broadcasted_iota(jnp.int32, (rows, cols), 1)
    P = jnp.zeros(x.shape, jnp.float32)
    for _ in range(k):
        m = jnp.max(x, axis=1, keepdims=True)
        cand = jnp.where(x == m, ci, cols)
        idx = jnp.min(cand, axis=1, keepdims=True)
        oh = ci == idx
        P = P + oh.astype(jnp.float32)
        x = jnp.where(oh, -jnp.inf, x)
    return P


def _tr(x_bf, eye_bf):
    return lax.dot_general(x_bf, eye_bf, (((0,), (0,)), ((), ())),
                           preferred_element_type=jnp.float32)


def _elu(x):
    return jnp.where(x > 0, x, jnp.exp(x) - 1.0)


def _pexp(z):
    e = jnp.maximum(z, ALPHA * z)
    return jnp.exp(jnp.minimum(e, 60.0))


def _att_chunk(a1_rows, a2_chunk, mask_chunk, h_chunk, ones_chunk):
    p = _pexp(a1_rows + a2_chunk) * mask_chunk
    num = lax.dot_general(p, h_chunk, (((1,), (0,)), ((), ())),
                          preferred_element_type=jnp.float32)
    s = lax.dot_general(p, ones_chunk, (((1,), (0,)), ((), ())),
                        preferred_element_type=jnp.float32)
    return num, s


def _body(hid_ref, ipro_ref, spro_ref, iemb_ref, semb_ref, W_ref, a_ref,
          Wout_ref, aout_ref, iW_ref, ib_ref, sW_ref, sb_ref,
          hidden_out, iout, sout, mtok_ref, mint_ref, mslot_ref):
    f32 = jnp.float32

    P_int = _topk_onehot(ipro_ref[0], TOPK)
    P_slot = _topk_onehot(spro_ref[0], TOPK)

    r512 = lax.broadcasted_iota(jnp.int32, (S, S), 0)
    c512 = lax.broadcasted_iota(jnp.int32, (S, S), 1)
    eyeS_bf = (r512 == c512).astype(jnp.bfloat16)
    P_int_bf = P_int.astype(jnp.bfloat16)
    P_slot_bf = P_slot.astype(jnp.bfloat16)

    @pl.when(pl.program_id(0) == 0)
    def _():
        mtok_ref[:, 0:S] = (jnp.abs(r512 - c512) <= WINDOW).astype(f32)
    rTI = lax.broadcasted_iota(jnp.int32, (S, INTENT), 0)
    cTI = lax.broadcasted_iota(jnp.int32, (S, INTENT), 1)
    spill = (cTI <= rTI - (S - WINDOW)).astype(f32)
    mtok_ref[:, S:SI] = jnp.maximum(P_int, spill)
    mtok_ref[:, SI:N] = P_slot
    mint_ref[:, 0:S] = _tr(P_int_bf, eyeS_bf)
    IS = lax.dot_general(P_int_bf, P_slot_bf, (((0,), (0,)), ((), ())),
                         preferred_element_type=f32)
    mint_ref[:, S:S + SLOT] = (IS > 0).astype(f32)
    mslot_ref[:, 0:S] = _tr(P_slot_bf, eyeS_bf)
    IST = lax.dot_general(P_slot_bf, P_int_bf, (((0,), (0,)), ((), ())),
                          preferred_element_type=f32)
    mslot_ref[:, S:SI] = (IST > 0).astype(f32)

    hcat = jnp.concatenate([hid_ref[0], iemb_ref[...], semb_ref[...]], axis=0)
    ones_n = jnp.ones((N, 1), jnp.float32)

    RT = 256
    heads = []
    for k in range(NHEAD):
        hk = jnp.dot(hcat, W_ref[k], preferred_element_type=f32)
        a1v = a_ref[k:k + 1, 0:GHD]
        a2v = a_ref[k:k + 1, GHD:2 * GHD]
        a1 = lax.dot_general(hk, a1v, (((1,), (1,)), ((), ())),
                             preferred_element_type=f32)
        a2c = lax.dot_general(hk, a2v, (((1,), (1,)), ((), ())),
                              preferred_element_type=f32)
        a2 = lax.dot_general(a2v, hk, (((1,), (1,)), ((), ())),
                             preferred_element_type=f32)
        parts = []
        for t in range(S // RT):
            num, s = _att_chunk(a1[t * RT:(t + 1) * RT], a2,
                                mtok_ref[t * RT:(t + 1) * RT, :], hk, ones_n)
            parts.append(_elu(num / s))
        a2_sel = jnp.concatenate([a2[:, 0:S], a2[:, SI:N]], axis=1)
        h_sel = jnp.concatenate([hk[0:S], hk[SI:N]], axis=0)
        num, s = _att_chunk(a1[S:SI], a2_sel, mint_ref[...], h_sel,
                            ones_n[0:S + SLOT])
        pd = _pexp(a1[S:SI] + a2c[S:SI])
        parts.append(_elu((num + pd * hk[S:SI]) / (s + pd)))
        num, s = _att_chunk(a1[SI:N], a2[:, 0:SI], mslot_ref[...], hk[0:SI],
                            ones_n[0:SI])
        pd = _pexp(a1[SI:N] + a2c[SI:N])
        parts.append(_elu((num + pd * hk[SI:N]) / (s + pd)))
        heads.append(jnp.concatenate(parts, axis=0))
    h1 = jnp.concatenate(heads, axis=1)

    h2 = jnp.dot(h1, Wout_ref[...], preferred_element_type=f32)
    a1o = lax.dot_general(h2, aout_ref[0:1, :], (((1,), (1,)), ((), ())),
                          preferred_element_type=f32)
    a2o = lax.dot_general(aout_ref[1:2, :], h2, (((1,), (1,)), ((), ())),
                          preferred_element_type=f32)
    for t in range(S // RT):
        num, s = _att_chunk(a1o[t * RT:(t + 1) * RT], a2o,
                            mtok_ref[t * RT:(t + 1) * RT, :], h2, ones_n)
        hid = _elu(num / s)
        hidden_out[0, t * RT:(t + 1) * RT, :] = hid
        iout[0, t * RT:(t + 1) * RT, :] = (
            jnp.dot(hid, iW_ref[...], preferred_element_type=f32) + ib_ref[...])
        sout[0, t * RT:(t + 1) * RT, :] = (
            jnp.dot(hid, sW_ref[...], preferred_element_type=f32) + sb_ref[...])


def kernel(hiddens, seq_lens, intent_pro, slot_pro, intent_embedding,
           slot_embedding, gat_W, gat_a, gat_Wout, gat_aout, intent_W,
           intent_b, slot_W, slot_b):
    del seq_lens
    aout2 = gat_aout.reshape(2, GOD)
    ib2 = intent_b.reshape(1, INTENT)
    sb2 = slot_b.reshape(1, SLOT)

    full = lambda shape: pl.BlockSpec(shape, lambda b: (0,) * len(shape))
    batched = lambda shape: pl.BlockSpec((1,) + shape, lambda b: (b, 0, 0))

    hidden, intent_out, slot_out = pl.pallas_call(
        _body,
        grid=(B,),
        in_specs=[
            batched((S, H)),
            batched((S, INTENT)),
            batched((S, SLOT)),
            full((INTENT, H)),
            full((SLOT, H)),
            full((NHEAD, H, GHD)),
            full((NHEAD, 2 * GHD)),
            full((NHEAD * GHD, GOD)),
            full((2, GOD)),
            full((GOD, INTENT)),
            full((1, INTENT)),
            full((GOD, SLOT)),
            full((1, SLOT)),
        ],
        out_specs=[
            batched((S, GOD)),
            batched((S, INTENT)),
            batched((S, SLOT)),
        ],
        out_shape=[
            jax.ShapeDtypeStruct((B, S, GOD), jnp.float32),
            jax.ShapeDtypeStruct((B, S, INTENT), jnp.float32),
            jax.ShapeDtypeStruct((B, S, SLOT), jnp.float32),
        ],
        scratch_shapes=[
            pltpu.VMEM((S, N), jnp.float32),
            pltpu.VMEM((INTENT, S + SLOT), jnp.float32),
            pltpu.VMEM((SLOT, SI), jnp.float32),
        ],
        compiler_params=pltpu.CompilerParams(
            dimension_semantics=("arbitrary",)),
    )(hiddens, intent_pro, slot_pro, intent_embedding, slot_embedding,
      gat_W, gat_a, gat_Wout, aout2, intent_W, ib2, slot_W, sb2)

    return (hidden, hidden, intent_out, slot_out)


# speedup vs baseline: 1.1140x; 1.1140x over previous
import jax
import jax.numpy as jnp
from jax import lax
from jax.experimental import pallas as pl
from jax.experimental.pallas import tpu as pltpu

B = 8; S = 512; H = 128; INTENT = 128; SLOT = 512
GHD = 16; GOD = 128; NHEAD = 4; TOPK = 3; WINDOW = 2; ALPHA = 0.2
N = S + INTENT + SLOT
SI = S + INTENT


def _topk_onehot(x, k):
    rows, cols = x.shape
    ci = lax.broadcasted_iota(jnp.int32, (rows, cols), 1)
    P = jnp.zeros(x.shape, jnp.float32)
    for _ in range(k):
        m = jnp.max(x, axis=1, keepdims=True)
        cand = jnp.where(x == m, ci, cols)
        idx = jnp.min(cand, axis=1, keepdims=True)
        oh = ci == idx
        P = P + oh.astype(jnp.float32)
        x = jnp.where(oh, -jnp.inf, x)
    return P


def _tr(x_bf, eye_bf):
    return lax.dot_general(x_bf, eye_bf, (((0,), (0,)), ((), ())),
                           preferred_element_type=jnp.float32)


def _elu(x):
    return jnp.where(x > 0, x, jnp.exp(x) - 1.0)


def _pexp(z):
    e = jnp.maximum(z, ALPHA * z)
    return jnp.exp(jnp.minimum(e, 60.0))


def _att_rows(a1_tile, a2_row, mask_tile, h, ones_col):
    p = _pexp(a1_tile + a2_row) * mask_tile
    num = lax.dot_general(p, h, (((1,), (0,)), ((), ())),
                          preferred_element_type=jnp.float32)
    s = lax.dot_general(p, ones_col, (((1,), (0,)), ((), ())),
                        preferred_element_type=jnp.float32)
    return num / s


def _body(hid_ref, ipro_ref, spro_ref, iemb_ref, semb_ref, W_ref, a_ref,
          Wout_ref, aout_ref, iW_ref, ib_ref, sW_ref, sb_ref,
          hidden_out, iout, sout, mask_ref):
    f32 = jnp.float32

    P_int = _topk_onehot(ipro_ref[0], TOPK)
    P_slot = _topk_onehot(spro_ref[0], TOPK)

    r512 = lax.broadcasted_iota(jnp.int32, (S, S), 0)
    c512 = lax.broadcasted_iota(jnp.int32, (S, S), 1)
    eyeS_bf = ((r512 == c512)).astype(jnp.bfloat16)
    P_int_bf = P_int.astype(jnp.bfloat16)
    P_slot_bf = P_slot.astype(jnp.bfloat16)

    @pl.when(pl.program_id(0) == 0)
    def _():
        mask_ref[0:S, 0:S] = (jnp.abs(r512 - c512) <= WINDOW).astype(f32)
        rI = lax.broadcasted_iota(jnp.int32, (INTENT, INTENT), 0)
        cI = lax.broadcasted_iota(jnp.int32, (INTENT, INTENT), 1)
        mask_ref[S:SI, S:SI] = (rI == cI).astype(f32)
        mask_ref[SI:N, SI:N] = (r512 == c512).astype(f32)

    rTI = lax.broadcasted_iota(jnp.int32, (S, INTENT), 0)
    cTI = lax.broadcasted_iota(jnp.int32, (S, INTENT), 1)
    spill = (cTI <= rTI - (S - WINDOW)).astype(f32)
    mask_ref[0:S, S:SI] = jnp.maximum(P_int, spill)
    mask_ref[0:S, SI:N] = P_slot
    mask_ref[S:SI, 0:S] = _tr(P_int_bf, eyeS_bf)
    IS = lax.dot_general(P_int_bf, P_slot_bf, (((0,), (0,)), ((), ())),
                         preferred_element_type=f32)
    mask_ref[S:SI, SI:N] = (IS > 0).astype(f32)
    mask_ref[SI:N, 0:S] = _tr(P_slot_bf, eyeS_bf)
    IST = lax.dot_general(P_slot_bf, P_int_bf, (((0,), (0,)), ((), ())),
                          preferred_element_type=f32)
    mask_ref[SI:N, S:SI] = (IST > 0).astype(f32)

    hcat = jnp.concatenate([hid_ref[0], iemb_ref[...], semb_ref[...]], axis=0)
    ones_n = jnp.ones((N, 1), f32)

    RT = 384
    heads = []
    for k in range(NHEAD):
        hk = jnp.dot(hcat, W_ref[k], preferred_element_type=f32)
        a1v = a_ref[k:k + 1, 0:GHD]
        a2v = a_ref[k:k + 1, GHD:2 * GHD]
        a1 = lax.dot_general(hk, a1v, (((1,), (1,)), ((), ())),
                             preferred_element_type=f32)
        a2 = lax.dot_general(a2v, hk, (((1,), (1,)), ((), ())),
                             preferred_element_type=f32)
        tiles = []
        for t in range(N // RT):
            o = _att_rows(a1[t * RT:(t + 1) * RT], a2,
                          mask_ref[t * RT:(t + 1) * RT, :], hk, ones_n)
            tiles.append(_elu(o))
        heads.append(jnp.concatenate(tiles, axis=0))
    h1 = jnp.concatenate(heads, axis=1)

    h2 = jnp.dot(h1, Wout_ref[...], preferred_element_type=f32)
    a1o = lax.dot_general(h2, aout_ref[0:1, :], (((1,), (1,)), ((), ())),
                          preferred_element_type=f32)
    a2o = lax.dot_general(aout_ref[1:2, :], h2, (((1,), (1,)), ((), ())),
                          preferred_element_type=f32)
    RT2 = 256
    for t in range(S // RT2):
        o = _att_rows(a1o[t * RT2:(t + 1) * RT2], a2o,
                      mask_ref[t * RT2:(t + 1) * RT2, :], h2, ones_n)
        hid = _elu(o)
        hidden_out[0, t * RT2:(t + 1) * RT2, :] = hid
        iout[0, t * RT2:(t + 1) * RT2, :] = (
            jnp.dot(hid, iW_ref[...], preferred_element_type=f32) + ib_ref[...])
        sout[0, t * RT2:(t + 1) * RT2, :] = (
            jnp.dot(hid, sW_ref[...], preferred_element_type=f32) + sb_ref[...])


def kernel(hiddens, seq_lens, intent_pro, slot_pro, intent_embedding,
           slot_embedding, gat_W, gat_a, gat_Wout, gat_aout, intent_W,
           intent_b, slot_W, slot_b):
    del seq_lens
    aout2 = gat_aout.reshape(2, GOD)
    ib2 = intent_b.reshape(1, INTENT)
    sb2 = slot_b.reshape(1, SLOT)

    full = lambda shape: pl.BlockSpec(shape, lambda b: (0,) * len(shape))
    batched = lambda shape: pl.BlockSpec((1,) + shape, lambda b: (b, 0, 0))

    hidden, intent_out, slot_out = pl.pallas_call(
        _body,
        grid=(B,),
        in_specs=[
            batched((S, H)),
            batched((S, INTENT)),
            batched((S, SLOT)),
            full((INTENT, H)),
            full((SLOT, H)),
            full((NHEAD, H, GHD)),
            full((NHEAD, 2 * GHD)),
            full((NHEAD * GHD, GOD)),
            full((2, GOD)),
            full((GOD, INTENT)),
            full((1, INTENT)),
            full((GOD, SLOT)),
            full((1, SLOT)),
        ],
        out_specs=[
            batched((S, GOD)),
            batched((S, INTENT)),
            batched((S, SLOT)),
        ],
        out_shape=[
            jax.ShapeDtypeStruct((B, S, GOD), jnp.float32),
            jax.ShapeDtypeStruct((B, S, INTENT), jnp.float32),
            jax.ShapeDtypeStruct((B, S, SLOT), jnp.float32),
        ],
        scratch_shapes=[pltpu.VMEM((N, N), jnp.float32)],
        compiler_params=pltpu.CompilerParams(
            dimension_semantics=("arbitrary",)),
    )(hiddens, intent_pro, slot_pro, intent_embedding, slot_embedding,
      gat_W, gat_a, gat_Wout, aout2, intent_W, ib2, slot_W, sb2)

    return (hidden, hidden, intent_out, slot_out)


# speedup vs baseline: 1.1628x; 1.0438x over previous
import jax
import jax.numpy as jnp
from jax import lax
from jax.experimental import pallas as pl
from jax.experimental.pallas import tpu as pltpu

B = 8; S = 512; H = 128; INTENT = 128; SLOT = 512
GHD = 16; GOD = 128; NHEAD = 4; TOPK = 3; WINDOW = 2; ALPHA = 0.2
N = S + INTENT + SLOT
SI = S + INTENT


def _topk_onehot(x, k):
    rows, cols = x.shape
    ci = lax.broadcasted_iota(jnp.int32, (rows, cols), 1)
    P = jnp.zeros(x.shape, jnp.float32)
    for _ in range(k):
        m = jnp.max(x, axis=1, keepdims=True)
        cand = jnp.where(x == m, ci, cols)
        idx = jnp.min(cand, axis=1, keepdims=True)
        oh = ci == idx
        P = P + oh.astype(jnp.float32)
        x = jnp.where(oh, -jnp.inf, x)
    return P


def _tr(x_bf, eye_bf):
    return lax.dot_general(x_bf, eye_bf, (((0,), (0,)), ((), ())),
                           preferred_element_type=jnp.float32)


def _elu(x):
    return jnp.where(x > 0, x, jnp.exp(x) - 1.0)


def _pexp(z):
    e = jnp.maximum(z, ALPHA * z)
    return jnp.exp(jnp.minimum(e, 60.0))


def _att_rows(a1_tile, a2_row, mask_tile, h, ones_col):
    p = _pexp(a1_tile + a2_row) * mask_tile
    num = lax.dot_general(p, h, (((1,), (0,)), ((), ())),
                          preferred_element_type=jnp.float32)
    s = lax.dot_general(p, ones_col, (((1,), (0,)), ((), ())),
                        preferred_element_type=jnp.float32)
    return num / s


def _body(hid_ref, ipro_ref, spro_ref, iemb_ref, semb_ref, W_ref, a_ref,
          Wout_ref, aout_ref, iW_ref, ib_ref, sW_ref, sb_ref,
          hidden_out, iout, sout, mask_ref):
    f32 = jnp.float32

    P_int = _topk_onehot(ipro_ref[0], TOPK)
    P_slot = _topk_onehot(spro_ref[0], TOPK)

    r512 = lax.broadcasted_iota(jnp.int32, (S, S), 0)
    c512 = lax.broadcasted_iota(jnp.int32, (S, S), 1)
    eyeS_bf = ((r512 == c512)).astype(jnp.bfloat16)
    P_int_bf = P_int.astype(jnp.bfloat16)
    P_slot_bf = P_slot.astype(jnp.bfloat16)

    mask_ref[0:S, 0:S] = (jnp.abs(r512 - c512) <= WINDOW).astype(f32)
    rI = lax.broadcasted_iota(jnp.int32, (INTENT, INTENT), 0)
    cI = lax.broadcasted_iota(jnp.int32, (INTENT, INTENT), 1)
    mask_ref[S:SI, S:SI] = (rI == cI).astype(f32)
    mask_ref[SI:N, SI:N] = (r512 == c512).astype(f32)

    rTI = lax.broadcasted_iota(jnp.int32, (S, INTENT), 0)
    cTI = lax.broadcasted_iota(jnp.int32, (S, INTENT), 1)
    spill = (cTI <= rTI - (S - WINDOW)).astype(f32)
    mask_ref[0:S, S:SI] = jnp.maximum(P_int, spill)
    mask_ref[0:S, SI:N] = P_slot
    mask_ref[S:SI, 0:S] = _tr(P_int_bf, eyeS_bf)
    IS = lax.dot_general(P_int_bf, P_slot_bf, (((0,), (0,)), ((), ())),
                         preferred_element_type=f32)
    mask_ref[S:SI, SI:N] = (IS > 0).astype(f32)
    mask_ref[SI:N, 0:S] = _tr(P_slot_bf, eyeS_bf)
    IST = lax.dot_general(P_slot_bf, P_int_bf, (((0,), (0,)), ((), ())),
                          preferred_element_type=f32)
    mask_ref[SI:N, S:SI] = (IST > 0).astype(f32)

    hcat = jnp.concatenate([hid_ref[0], iemb_ref[...], semb_ref[...]], axis=0)
    ones_n = jnp.ones((N, 1), f32)

    RT = 384
    heads = []
    for k in range(NHEAD):
        hk = jnp.dot(hcat, W_ref[k], preferred_element_type=f32)
        a1v = a_ref[k:k + 1, 0:GHD]
        a2v = a_ref[k:k + 1, GHD:2 * GHD]
        a1 = lax.dot_general(hk, a1v, (((1,), (1,)), ((), ())),
                             preferred_element_type=f32)
        a2 = lax.dot_general(a2v, hk, (((1,), (1,)), ((), ())),
                             preferred_element_type=f32)
        tiles = []
        for t in range(N // RT):
            o = _att_rows(a1[t * RT:(t + 1) * RT], a2,
                          mask_ref[t * RT:(t + 1) * RT, :], hk, ones_n)
            tiles.append(_elu(o))
        heads.append(jnp.concatenate(tiles, axis=0))
    h1 = jnp.concatenate(heads, axis=1)

    h2 = jnp.dot(h1, Wout_ref[...], preferred_element_type=f32)
    a1o = lax.dot_general(h2, aout_ref[0:1, :], (((1,), (1,)), ((), ())),
                          preferred_element_type=f32)
    a2o = lax.dot_general(aout_ref[1:2, :], h2, (((1,), (1,)), ((), ())),
                          preferred_element_type=f32)
    RT2 = 256
    for t in range(S // RT2):
        o = _att_rows(a1o[t * RT2:(t + 1) * RT2], a2o,
                      mask_ref[t * RT2:(t + 1) * RT2, :], h2, ones_n)
        hid = _elu(o)
        hidden_out[0, t * RT2:(t + 1) * RT2, :] = hid
        iout[0, t * RT2:(t + 1) * RT2, :] = (
            jnp.dot(hid, iW_ref[...], preferred_element_type=f32) + ib_ref[...])
        sout[0, t * RT2:(t + 1) * RT2, :] = (
            jnp.dot(hid, sW_ref[...], preferred_element_type=f32) + sb_ref[...])


def kernel(hiddens, seq_lens, intent_pro, slot_pro, intent_embedding,
           slot_embedding, gat_W, gat_a, gat_Wout, gat_aout, intent_W,
           intent_b, slot_W, slot_b):
    del seq_lens
    aout2 = gat_aout.reshape(2, GOD)
    ib2 = intent_b.reshape(1, INTENT)
    sb2 = slot_b.reshape(1, SLOT)

    full = lambda shape: pl.BlockSpec(shape, lambda b: (0,) * len(shape))
    batched = lambda shape: pl.BlockSpec((1,) + shape, lambda b: (b, 0, 0))

    hidden, intent_out, slot_out = pl.pallas_call(
        _body,
        grid=(B,),
        in_specs=[
            batched((S, H)),
            batched((S, INTENT)),
            batched((S, SLOT)),
            full((INTENT, H)),
            full((SLOT, H)),
            full((NHEAD, H, GHD)),
            full((NHEAD, 2 * GHD)),
            full((NHEAD * GHD, GOD)),
            full((2, GOD)),
            full((GOD, INTENT)),
            full((1, INTENT)),
            full((GOD, SLOT)),
            full((1, SLOT)),
        ],
        out_specs=[
            batched((S, GOD)),
            batched((S, INTENT)),
            batched((S, SLOT)),
        ],
        out_shape=[
            jax.ShapeDtypeStruct((B, S, GOD), jnp.float32),
            jax.ShapeDtypeStruct((B, S, INTENT), jnp.float32),
            jax.ShapeDtypeStruct((B, S, SLOT), jnp.float32),
        ],
        scratch_shapes=[pltpu.VMEM((N, N), jnp.float32)],
        compiler_params=pltpu.CompilerParams(
            dimension_semantics=("arbitrary",)),
    )(hiddens, intent_pro, slot_pro, intent_embedding, slot_embedding,
      gat_W, gat_a, gat_Wout, aout2, intent_W, ib2, slot_W, sb2)

    return (hidden, hidden, intent_out, slot_out)
